# baseline (speedup 1.0000x reference)
import functools

import jax
import jax.numpy as jnp
from jax import lax
from jax.experimental import pallas as pl
from jax.experimental.pallas import tpu as pltpu
from jax.experimental.pallas import tpu_sc as plsc

NPTS = 500
PAD = 512
CF = 160
DIM = 64
NS = 16
PPT = PAD // NS
NCH = CF // 16


def _sc_body(us, u1, u2, i8, i4, i2, i14, wv, b1, dfft, dfst, plus2_out,
             idx8_v, idx4_v, idx2_v, idx14_v, wv_v, b1_v,
             usl, dffl, dfsl, rows0, rows1, e_acc, h_acc,
             hsh, dssh, nfsh, sem0, sem1, sem2):
    wid = lax.axis_index("s")
    base = wid * PPT

    c_i8 = pltpu.async_copy(i8.at[pl.ds(base * 8, PPT * 8)], idx8_v, sem0)
    c_i4 = pltpu.async_copy(i4.at[pl.ds(base * 4, PPT * 4)], idx4_v, sem0)
    c_i2 = pltpu.async_copy(i2.at[pl.ds(base * 8, PPT * 8)], idx2_v, sem0)
    c_i14 = pltpu.async_copy(i14.at[pl.ds(base * 4, PPT * 4)], idx14_v, sem0)
    c_wv = pltpu.async_copy(wv.at[pl.ds(base * 8, PPT * 8)], wv_v, sem0)
    c_b1 = pltpu.async_copy(b1, b1_v, sem0)
    c_us = pltpu.async_copy(us.at[pl.ds(base, PPT)], usl, sem0)
    c_dff = pltpu.async_copy(dfft.at[pl.ds(base, PPT)], dffl, sem0)
    c_dfs = pltpu.async_copy(dfst.at[pl.ds(base, PPT)], dfsl, sem0)
    c_i8.wait()

    ga0 = pltpu.async_copy(u1.at[idx8_v.at[pl.ds(0, 128)]], rows0, sem1)
    ga1 = pltpu.async_copy(u1.at[idx8_v.at[pl.ds(128, 128)]], rows1, sem2)
    c_wv.wait()
    c_us.wait()

    def acc_stage(rows, ch, acc, init):
        def acc_body(q, carry):
            wvec = wv_v[pl.ds(ch * 128 + q * 16, 16)] * 0.125
            for t in range(2):
                p = q * 2 + t
                g = ch * 16 + p
                ws = [wvec[t * 8 + j] for j in range(8)]
                for c in range(NCH):
                    sl = pl.ds(c * 16, 16)
                    v = usl[g, sl] if init else acc[g, sl]
                    for j in range(8):
                        v = v - ws[j] * rows[p * 8 + j, sl]
                    acc[g, sl] = v
            return carry
        lax.fori_loop(0, 8, acc_body, 0)

    ga0.wait()
    acc_stage(rows0, 0, e_acc, True)
    ga1.wait()
    gb0 = pltpu.async_copy(u2.at[idx8_v.at[pl.ds(0, 128)]], rows0, sem1)
    acc_stage(rows1, 1, e_acc, True)
    gb1 = pltpu.async_copy(u2.at[idx8_v.at[pl.ds(128, 128)]], rows1, sem2)
    gb0.wait()
    acc_stage(rows0, 0, h_acc, True)
    gb1.wait()
    acc_stage(rows1, 1, h_acc, True)

    pltpu.sync_copy(h_acc, hsh.at[pl.ds(base, PPT)])
    plsc.subcore_barrier()
    c_i4.wait()
    pltpu.async_copy(hsh.at[idx4_v], rows0, sem1).wait()
    c_b1.wait()

    def sig_body(p, carry):
        for c in range(NCH):
            sl = pl.ds(c * 16, 16)
            s = rows0[4 * p, sl] + rows0[4 * p + 1, sl]
            s = s + rows0[4 * p + 2, sl] + rows0[4 * p + 3, sl]
            x = e_acc[p, sl] + s * 0.25 + b1_v[sl]
            h_acc[p, sl] = 1.0 / (1.0 + jnp.exp(-x))
        return carry

    lax.fori_loop(0, PPT, sig_body, 0)

    pltpu.sync_copy(h_acc, dssh.at[pl.ds(base, PPT)])
    plsc.subcore_barrier()
    c_i2.wait()
    gd0 = pltpu.async_copy(dssh.at[idx2_v.at[pl.ds(0, 128)]], rows0, sem1)
    gd1 = pltpu.async_copy(dssh.at[idx2_v.at[pl.ds(128, 128)]], rows1, sem2)
    c_dff.wait()

    def times_stage(rows, ch):
        def times_body(p, carry):
            g = ch * 16 + p
            for c in range(NCH):
                sl = pl.ds(c * 16, 16)
                s = rows[p * 8, sl] + rows[p * 8 + 1, sl]
                for j in range(2, 8):
                    s = s + rows[p * 8 + j, sl]
                e_acc[g, sl] = dffl[g, sl] * s * 0.125
            return carry
        lax.fori_loop(0, 16, times_body, 0)

    gd0.wait()
    times_stage(rows0, 0)
    gd1.wait()
    times_stage(rows1, 1)

    pltpu.sync_copy(e_acc, nfsh.at[pl.ds(base, PPT)])
    plsc.subcore_barrier()
    c_i14.wait()
    pltpu.async_copy(nfsh.at[idx14_v], rows0, sem1).wait()
    c_dfs.wait()

    def plus2_body(p, carry):
        for c in range(NCH):
            sl = pl.ds(c * 16, 16)
            s = rows0[4 * p, sl] + rows0[4 * p + 1, sl]
            s = s + rows0[4 * p + 2, sl] + rows0[4 * p + 3, sl]
            h_acc[p, sl] = dfsl[p, sl] + s * 0.25
        return carry

    lax.fori_loop(0, PPT, plus2_body, 0)
    pltpu.sync_copy(h_acc, plus2_out.at[pl.ds(base, PPT)])


@functools.lru_cache(maxsize=None)
def _sc_kernel():
    mesh = plsc.VectorSubcoreMesh(
        core_axis_name="c", subcore_axis_name="s", num_cores=1,
        num_subcores=NS)
    return pl.kernel(
        _sc_body,
        out_type=jax.ShapeDtypeStruct((PAD, CF), jnp.float32),
        mesh=mesh,
        compiler_params=pltpu.CompilerParams(use_tc_tiling_on_sc=False),
        scratch_types=[
            pltpu.VMEM((PPT * 8,), jnp.int32),
            pltpu.VMEM((PPT * 4,), jnp.int32),
            pltpu.VMEM((PPT * 8,), jnp.int32),
            pltpu.VMEM((PPT * 4,), jnp.int32),
            pltpu.VMEM((PPT * 8,), jnp.float32),
            pltpu.VMEM((CF,), jnp.float32),
            pltpu.VMEM((PPT, CF), jnp.float32),
            pltpu.VMEM((PPT, CF), jnp.float32),
            pltpu.VMEM((PPT, CF), jnp.float32),
            pltpu.VMEM((128, CF), jnp.float32),
            pltpu.VMEM((128, CF), jnp.float32),
            pltpu.VMEM((PPT, CF), jnp.float32),
            pltpu.VMEM((PPT, CF), jnp.float32),
            pltpu.VMEM_SHARED((PAD, CF), jnp.float32),
            pltpu.VMEM_SHARED((PAD, CF), jnp.float32),
            pltpu.VMEM_SHARED((PAD, CF), jnp.float32),
            pltpu.SemaphoreType.DMA,
            pltpu.SemaphoreType.DMA,
            pltpu.SemaphoreType.DMA,
        ],
    )


def _tc_pre_body(ft_ref, f1t_ref, f2t_ref, wvr_ref, w1t_ref,
                 us_ref, u1_ref, u2_ref):
    w1t = w1t_ref[...]
    s1 = jnp.sum(wvr_ref[...], axis=1, keepdims=True) * 0.125
    us_ref[...] = jnp.dot(ft_ref[...], w1t,
                          preferred_element_type=jnp.float32) * s1
    u1_ref[...] = jnp.dot(f1t_ref[...], w1t, preferred_element_type=jnp.float32)
    u2_ref[...] = jnp.dot(f2t_ref[...], w1t, preferred_element_type=jnp.float32)


def _tc_conv2_body(plus2_ref, w3_ref, b3_ref, out_ref):
    out_ref[...] = lax.dot_general(
        w3_ref[...], plus2_ref[...], (((1,), (1,)), ((), ())),
        preferred_element_type=jnp.float32) + b3_ref[...]


@jax.jit
def kernel(feat, feat1, feat2, inds, inds1, inds2, wei1, wei2,
           dens_feat_f, dens_feat_s, W1, b1, W3, b3):
    del wei2
    padp = PAD - NPTS

    def padt(x):
        return jnp.pad(x[0].T, ((0, padp), (0, 0)))

    def padflat(x, k):
        return jnp.pad(x[0].astype(jnp.int32).reshape(NPTS, k),
                       ((0, padp), (0, 0))).reshape(-1)

    ft = padt(feat)
    f1t = padt(feat1)
    f2t = padt(feat2)
    dfft = padt(dens_feat_f)
    dfst = padt(dens_feat_s)
    i8 = padflat(inds1, 8)
    i4 = padflat(inds, 4)
    i2 = padflat(inds2, 8)
    i14 = jnp.pad(inds1[0].astype(jnp.int32).reshape(NPTS, 8)[:, :4],
                  ((0, padp), (0, 0))).reshape(-1)
    wvr = jnp.pad(wei1[0].reshape(NPTS, 8), ((0, padp), (0, 0)))

    us_t, u1_t, u2_t = pl.pallas_call(
        _tc_pre_body,
        out_shape=[jax.ShapeDtypeStruct((PAD, CF), jnp.float32)] * 3,
    )(ft, f1t, f2t, wvr, W1.T)

    plus2_t = _sc_kernel()(us_t, u1_t, u2_t, i8, i4, i2, i14,
                           wvr.reshape(-1), b1, dfft, dfst)

    out = pl.pallas_call(
        _tc_conv2_body,
        out_shape=jax.ShapeDtypeStruct((CF, PAD), jnp.float32),
    )(plus2_t, W3, b3[:, None])
    return out[None, :, :NPTS]


# speedup vs baseline: 3.9006x; 3.9006x over previous
import jax
import jax.numpy as jnp
from jax import lax
from jax.experimental import pallas as pl

NPTS = 500
PAD = 512
CF = 160
DIM = 64


def _fused_body(f_ref, f1_ref, f2_ref, dff_ref, dfs_ref,
                inds1_ref, inds_ref, inds2_ref, wei1_ref,
                w1_ref, b1_ref, w3_ref, b3_ref, out_ref):
    iota = lax.broadcasted_iota(jnp.int32, (PAD, PAD), 1)
    cdims = (((1,), (1,)), ((), ()))

    inds1 = inds1_ref[...]
    wei1 = wei1_ref[...] * 0.125
    at = jnp.zeros((PAD, PAD), jnp.float32)
    dt = jnp.zeros((PAD, PAD), jnp.float32)
    for j in range(8):
        mask = inds1[:, j:j + 1] == iota
        at = at + jnp.where(mask, wei1[:, j:j + 1], 0.0)
        if j < 4:
            dt = dt + jnp.where(mask, 0.25, 0.0)
    inds = inds_ref[...]
    bt = jnp.zeros((PAD, PAD), jnp.float32)
    for j in range(4):
        bt = bt + jnp.where(inds[:, j:j + 1] == iota, 0.25, 0.0)
    inds2 = inds2_ref[...]
    ct = jnp.zeros((PAD, PAD), jnp.float32)
    for j in range(8):
        ct = ct + jnp.where(inds2[:, j:j + 1] == iota, 0.125, 0.0)

    s1row = lax.dot_general(jnp.ones((1, PAD), jnp.float32), at, cdims,
                            preferred_element_type=jnp.float32)

    f = f_ref[...]
    fs1 = f * s1row
    pix = fs1 - lax.dot_general(f1_ref[...], at, cdims,
                                preferred_element_type=jnp.float32)
    pt = fs1 - lax.dot_general(f2_ref[...], at, cdims,
                               preferred_element_type=jnp.float32)
    plus = pix + lax.dot_general(pt, bt, cdims,
                                 preferred_element_type=jnp.float32)
    ds = jax.nn.sigmoid(
        jnp.dot(w1_ref[...], plus, preferred_element_type=jnp.float32)
        + b1_ref[...])
    m = lax.dot_general(ds, ct, cdims, preferred_element_type=jnp.float32)
    new_f = dff_ref[...] * m
    plus2 = dfs_ref[...] + lax.dot_general(
        new_f, dt, cdims, preferred_element_type=jnp.float32)
    out_ref[...] = (jnp.dot(w3_ref[...], plus2,
                            preferred_element_type=jnp.float32) + b3_ref[...])


@jax.jit
def kernel(feat, feat1, feat2, inds, inds1, inds2, wei1, wei2,
           dens_feat_f, dens_feat_s, W1, b1, W3, b3):
    del wei2
    padp = PAD - NPTS

    def padc(x):
        return jnp.pad(x[0], ((0, 0), (0, padp)))

    def padi(x, k):
        return jnp.pad(x[0].astype(jnp.int32).reshape(NPTS, k),
                       ((0, padp), (0, 0)))

    out = pl.pallas_call(
        _fused_body,
        out_shape=jax.ShapeDtypeStruct((CF, PAD), jnp.float32),
    )(padc(feat), padc(feat1), padc(feat2),
      padc(dens_feat_f), padc(dens_feat_s),
      padi(inds1, 8), padi(inds, 4), padi(inds2, 8),
      jnp.pad(wei1[0].reshape(NPTS, 8), ((0, padp), (0, 0))),
      W1, b1[:, None], W3, b3[:, None])
    return out[None, :, :NPTS]


# speedup vs baseline: 4.7246x; 1.2112x over previous
import jax
import jax.numpy as jnp
from jax import lax
from jax.experimental import pallas as pl

NPTS = 500
PAD = 512
CF = 160
DIM = 64


def _fused_body(f_ref, f1_ref, f2_ref, dff_ref, dfs_ref,
                inds1_ref, inds_ref, inds2_ref, wei1_ref,
                w1_ref, b1_ref, w3_ref, b3_ref, out_ref):
    iota = lax.broadcasted_iota(jnp.int32, (PAD, PAD), 1)
    cdims = (((1,), (1,)), ((), ()))
    padp = PAD - NPTS

    def padc(x):
        return jnp.pad(x, ((0, 0), (0, padp)))

    def rows(ref, k):
        del k
        return jnp.pad(ref[...], ((0, padp), (0, 0)))

    inds1 = rows(inds1_ref, 8)
    wei1 = rows(wei1_ref, 8) * 0.125
    at = jnp.zeros((PAD, PAD), jnp.float32)
    dt = jnp.zeros((PAD, PAD), jnp.float32)
    for j in range(8):
        mask = inds1[:, j:j + 1] == iota
        at = at + jnp.where(mask, wei1[:, j:j + 1], 0.0)
        if j < 4:
            dt = dt + jnp.where(mask, 0.25, 0.0)
    inds = rows(inds_ref, 4)
    bt = jnp.zeros((PAD, PAD), jnp.float32)
    for j in range(4):
        bt = bt + jnp.where(inds[:, j:j + 1] == iota, 0.25, 0.0)
    inds2 = rows(inds2_ref, 8)
    ct = jnp.zeros((PAD, PAD), jnp.float32)
    for j in range(8):
        ct = ct + jnp.where(inds2[:, j:j + 1] == iota, 0.125, 0.0)

    s1row = lax.dot_general(jnp.ones((1, PAD), jnp.float32), at, cdims,
                            preferred_element_type=jnp.float32)

    f = padc(f_ref[...])
    fs1 = f * s1row
    pix = fs1 - lax.dot_general(padc(f1_ref[...]), at, cdims,
                                preferred_element_type=jnp.float32)
    pt = fs1 - lax.dot_general(padc(f2_ref[...]), at, cdims,
                               preferred_element_type=jnp.float32)
    plus = pix + lax.dot_general(pt, bt, cdims,
                                 preferred_element_type=jnp.float32)
    ds = jax.nn.sigmoid(
        jnp.dot(w1_ref[...], plus, preferred_element_type=jnp.float32)
        + b1_ref[...][:, None])
    m = lax.dot_general(ds, ct, cdims, preferred_element_type=jnp.float32)
    new_f = padc(dff_ref[...]) * m
    plus2 = padc(dfs_ref[...]) + lax.dot_general(
        new_f, dt, cdims, preferred_element_type=jnp.float32)
    out_ref[...] = (jnp.dot(w3_ref[...], plus2,
                            preferred_element_type=jnp.float32)
                    + b3_ref[...][:, None])


@jax.jit
def kernel(feat, feat1, feat2, inds, inds1, inds2, wei1, wei2,
           dens_feat_f, dens_feat_s, W1, b1, W3, b3):
    del wei2
    out = pl.pallas_call(
        _fused_body,
        out_shape=jax.ShapeDtypeStruct((CF, PAD), jnp.float32),
    )(feat[0], feat1[0], feat2[0],
      dens_feat_f[0], dens_feat_s[0],
      inds1[0].astype(jnp.int32).reshape(NPTS, 8),
      inds[0].astype(jnp.int32).reshape(NPTS, 4),
      inds2[0].astype(jnp.int32).reshape(NPTS, 8),
      wei1[0].reshape(NPTS, 8),
      W1, b1, W3, b3)
    return out[None, :, :NPTS]


# speedup vs baseline: 5.4630x; 1.1563x over previous
import jax
import jax.numpy as jnp
from jax import lax
from jax.experimental import pallas as pl

NPTS = 500
PAD = 512
CF = 160
DIM = 64


def _fused_body(f_ref, f1_ref, f2_ref, dff_ref, dfs_ref,
                inds1_ref, inds_ref, inds2_ref, wei1_ref,
                w1_ref, b1_ref, w3_ref, b3_ref, out_ref):
    iota = lax.broadcasted_iota(jnp.int32, (PAD, PAD), 0)
    padp = PAD - NPTS

    def padc(x):
        return jnp.pad(x, ((0, 0), (0, padp)))

    def jrow(arr, j, k, fill):
        del k
        return jnp.pad(arr[j:j + 1, :], ((0, 0), (0, padp)),
                       constant_values=fill)

    inds1 = inds1_ref[...]
    wei1 = wei1_ref[...] * 0.125
    at = jnp.zeros((PAD, PAD), jnp.float32)
    dt = jnp.zeros((PAD, PAD), jnp.float32)
    s1row = jnp.zeros((1, PAD), jnp.float32)
    for j in range(8):
        mask = jrow(inds1, j, 8, -1) == iota
        wj = jrow(wei1, j, 8, 0.0)
        s1row = s1row + wj
        at = at + jnp.where(mask, wj, 0.0)
        if j < 4:
            dt = dt + jnp.where(mask, 0.25, 0.0)
    inds = inds_ref[...]
    bt = jnp.zeros((PAD, PAD), jnp.float32)
    for j in range(4):
        bt = bt + jnp.where(jrow(inds, j, 4, -1) == iota, 0.25, 0.0)
    inds2 = inds2_ref[...]
    ct = jnp.zeros((PAD, PAD), jnp.float32)
    for j in range(8):
        ct = ct + jnp.where(jrow(inds2, j, 8, -1) == iota, 0.125, 0.0)

    f = padc(f_ref[...])
    fs1 = f * s1row
    pix = fs1 - jnp.dot(padc(f1_ref[...]), at,
                        preferred_element_type=jnp.float32)
    pt = fs1 - jnp.dot(padc(f2_ref[...]), at,
                       preferred_element_type=jnp.float32)
    plus = pix + jnp.dot(pt, bt, preferred_element_type=jnp.float32)
    ds = jax.nn.sigmoid(
        jnp.dot(w1_ref[...], plus, preferred_element_type=jnp.float32)
        + b1_ref[...][:, None])
    m = jnp.dot(ds, ct, preferred_element_type=jnp.float32)
    new_f = padc(dff_ref[...]) * m
    plus2 = padc(dfs_ref[...]) + jnp.dot(new_f, dt,
                                         preferred_element_type=jnp.float32)
    res = (jnp.dot(w3_ref[...], plus2, preferred_element_type=jnp.float32)
           + b3_ref[...][:, None])
    out_ref[...] = res[None, :, :NPTS]


@jax.jit
def kernel(feat, feat1, feat2, inds, inds1, inds2, wei1, wei2,
           dens_feat_f, dens_feat_s, W1, b1, W3, b3):
    del wei2
    return pl.pallas_call(
        _fused_body,
        out_shape=jax.ShapeDtypeStruct((1, CF, NPTS), jnp.float32),
    )(feat[0], feat1[0], feat2[0],
      dens_feat_f[0], dens_feat_s[0],
      inds1[0].astype(jnp.int32).reshape(NPTS, 8).T,
      inds[0].astype(jnp.int32).reshape(NPTS, 4).T,
      inds2[0].astype(jnp.int32).reshape(NPTS, 8).T,
      wei1[0].reshape(NPTS, 8).T,
      W1, b1, W3, b3)


# speedup vs baseline: 6.7675x; 1.2388x over previous
import jax
import jax.numpy as jnp
from jax import lax
from jax.experimental import pallas as pl

NPTS = 500
PAD = 512
CF = 160
DIM = 64


def _fused_body(f_ref, f1_ref, f2_ref, dff_ref, dfs_ref, comb_ref,
                w1_ref, b1_ref, w3_ref, b3_ref, out_ref):
    iota = lax.broadcasted_iota(jnp.int32, (PAD, PAD), 0)
    padp = PAD - NPTS

    def padc(x):
        return jnp.pad(x, ((0, 0), (0, padp)))

    comb = comb_ref[...]

    def jrow(j, off, fill):
        return jnp.pad(comb[j:j + 1, off:off + NPTS], ((0, 0), (0, padp)),
                       constant_values=fill)

    at = jnp.zeros((PAD, PAD), jnp.float32)
    dt = jnp.zeros((PAD, PAD), jnp.float32)
    ct = jnp.zeros((PAD, PAD), jnp.float32)
    bt = jnp.zeros((PAD, PAD), jnp.float32)
    s1row = jnp.zeros((1, PAD), jnp.float32)
    for j in range(8):
        mask = jrow(j, 0, -1) == iota
        wj = lax.bitcast_convert_type(jrow(j, 1000, 0), jnp.float32) * 0.125
        s1row = s1row + wj
        at = at + jnp.where(mask, wj, 0.0)
        if j < 4:
            dt = dt + jnp.where(mask, 0.25, 0.0)
            bt = bt + jnp.where(jrow(j, 1500, -1) == iota, 0.25, 0.0)
        ct = ct + jnp.where(jrow(j, 500, -1) == iota, 0.125, 0.0)

    f = padc(f_ref[...])
    fs1 = f * s1row
    pix = fs1 - jnp.dot(padc(f1_ref[...]), at,
                        preferred_element_type=jnp.float32)
    pt = fs1 - jnp.dot(padc(f2_ref[...]), at,
                       preferred_element_type=jnp.float32)
    plus = pix + jnp.dot(pt, bt, preferred_element_type=jnp.float32)
    ds = jax.nn.sigmoid(
        jnp.dot(w1_ref[...], plus, preferred_element_type=jnp.float32)
        + b1_ref[...][:, None])
    m = jnp.dot(ds, ct, preferred_element_type=jnp.float32)
    new_f = padc(dff_ref[...]) * m
    plus2 = padc(dfs_ref[...]) + jnp.dot(new_f, dt,
                                         preferred_element_type=jnp.float32)
    res = (jnp.dot(w3_ref[...], plus2, preferred_element_type=jnp.float32)
           + b3_ref[...][:, None])
    out_ref[...] = res[None, :, :NPTS]


@jax.jit
def kernel(feat, feat1, feat2, inds, inds1, inds2, wei1, wei2,
           dens_feat_f, dens_feat_s, W1, b1, W3, b3):
    del wei2
    comb = jnp.concatenate([
        inds1[0].astype(jnp.int32).reshape(NPTS, 8).T,
        inds2[0].astype(jnp.int32).reshape(NPTS, 8).T,
        lax.bitcast_convert_type(wei1[0], jnp.int32).reshape(NPTS, 8).T,
        jnp.pad(inds[0].astype(jnp.int32).reshape(NPTS, 4).T,
                ((0, 4), (0, 0))),
    ], axis=1)
    return pl.pallas_call(
        _fused_body,
        out_shape=jax.ShapeDtypeStruct((1, CF, NPTS), jnp.float32),
    )(feat[0], feat1[0], feat2[0],
      dens_feat_f[0], dens_feat_s[0], comb,
      W1, b1, W3, b3)
